# baseline (speedup 1.0000x reference)
import functools

import jax
import jax.numpy as jnp
from jax import lax
from jax.experimental import pallas as pl
from jax.experimental.pallas import tpu as pltpu
from jax.experimental.pallas import tpu_sc as plsc

BATCH = 16384
D_UB = 64
D_G = 32
NC, NS = 2, 16
NW = NC * NS
ROWS_PER_W = BATCH // NW
CHUNK = 128
NCHUNK = ROWS_PER_W // CHUNK


def _pack_body(in_ref, out_ref):
    x = in_ref[...]
    d, bc = x.shape
    k = 128 // d
    xt = x.T
    pieces = []
    for g in range(bc // (128 * k)):
        pieces.append(jnp.concatenate(
            [xt[(g * k + h) * 128:(g * k + h + 1) * 128, :]
             for h in range(k)], axis=1))
    out_ref[...] = jnp.concatenate(pieces, axis=0)


def _tc_pack(tab_t, bc):
    d, n = tab_t.shape
    grid = (n + bc - 1) // bc
    rows = grid * (bc * d // 128)
    return pl.pallas_call(
        _pack_body,
        grid=(grid,),
        in_specs=[pl.BlockSpec((d, bc), lambda i: (0, i))],
        out_specs=pl.BlockSpec((bc * d // 128, 128), lambda i: (i, 0)),
        out_shape=jax.ShapeDtypeStruct((rows, 128), jnp.float32),
    )(tab_t)


def _sc_gather(user_idx, book_idx, genre_idx, up2, bp2, gp4):
    mesh = plsc.VectorSubcoreMesh(
        core_axis_name="c", subcore_axis_name="s", num_cores=NC, num_subcores=NS
    )

    @functools.partial(
        pl.kernel,
        mesh=mesh,
        out_type=[
            jax.ShapeDtypeStruct((BATCH, 128), jnp.float32),
            jax.ShapeDtypeStruct((BATCH, 128), jnp.float32),
            jax.ShapeDtypeStruct((BATCH, 128), jnp.float32),
        ],
        scratch_types=[
            pltpu.VMEM((ROWS_PER_W,), jnp.int32),
            pltpu.VMEM((ROWS_PER_W,), jnp.int32),
            pltpu.VMEM((ROWS_PER_W, 128), jnp.float32),
            pltpu.SemaphoreType.DMA,
        ],
    )
    def gather_kernel(uidx_hbm, bidx_hbm, gidx_hbm, up2_hbm, bp2_hbm, gp4_hbm,
                      uout_hbm, bout_hbm, gout_hbm,
                      idx_v, pidx_v, rows_a, sem):
        wid = lax.axis_index("s") * NC + lax.axis_index("c")
        base = wid * ROWS_PER_W

        def run(idx_hbm, tab_hbm, out_hbm, shift, rows_v):
            pltpu.sync_copy(idx_hbm.at[pl.ds(base, ROWS_PER_W)], idx_v)
            for k in range(ROWS_PER_W // 16):
                s = pl.ds(k * 16, 16)
                i = idx_v[s]
                pidx_v[s] = lax.shift_left(
                    lax.shift_right_logical(i, shift), 7) | (i & 127)
            copies = []
            for j in range(NCHUNK):
                s = pl.ds(j * CHUNK, CHUNK)
                copies.append(pltpu.async_copy(
                    tab_hbm.at[pidx_v.at[s]], rows_v.at[s], sem))
            for c in copies:
                c.wait()
            pltpu.sync_copy(rows_v, out_hbm.at[pl.ds(base, ROWS_PER_W)])

        run(uidx_hbm, up2_hbm, uout_hbm, 8, rows_a)
        run(bidx_hbm, bp2_hbm, bout_hbm, 8, rows_a)
        run(gidx_hbm, gp4_hbm, gout_hbm, 9, rows_a)

    return gather_kernel(user_idx, book_idx, genre_idx, up2, bp2, gp4)


def _mlp_body(ue_ref, be_ref, ge_ref, mu_ref, mb_ref, qg_ref,
              w1u_ref, w1b_ref, w1g_ref, b1_ref, w2_ref, b2_ref, w3_ref,
              b3_ref, out_ref):
    mu = mu_ref[...] > 0.5
    mb = mb_ref[...] > 0.5
    q0 = qg_ref[:, 0:1] > 0.5
    q1 = qg_ref[:, 1:2] > 0.5
    ue = ue_ref[...]
    be = be_ref[...]
    ge = ge_ref[...]
    xu = jnp.where(mu, ue[:, 64:], ue[:, :64])
    xb = jnp.where(mb, be[:, 64:], be[:, :64])
    xg = jnp.where(
        q1,
        jnp.where(q0, ge[:, 96:128], ge[:, 64:96]),
        jnp.where(q0, ge[:, 32:64], ge[:, 0:32]),
    )
    h1 = (xu @ w1u_ref[...] + xb @ w1b_ref[...] + xg @ w1g_ref[...]
          + b1_ref[...])
    h1 = jnp.maximum(h1, 0.0)
    h2 = jnp.maximum(h1 @ w2_ref[...] + b2_ref[...], 0.0)
    out_ref[...] = h2 @ w3_ref[...] + b3_ref[0]


def _tc_mlp(ue, be, ge, mu, mb, qg, W1, b1, W2, b2, W3, b3, bsz=4096):
    W1u, W1b, W1g = W1[:D_UB], W1[D_UB:2 * D_UB], W1[2 * D_UB:]
    grid = BATCH // bsz
    fixed = lambda *shape: pl.BlockSpec(shape, lambda i: (0,) * len(shape))
    out = pl.pallas_call(
        _mlp_body,
        grid=(grid,),
        in_specs=[
            pl.BlockSpec((bsz, 128), lambda i: (i, 0)),
            pl.BlockSpec((bsz, 128), lambda i: (i, 0)),
            pl.BlockSpec((bsz, 128), lambda i: (i, 0)),
            pl.BlockSpec((bsz, 1), lambda i: (i, 0)),
            pl.BlockSpec((bsz, 1), lambda i: (i, 0)),
            pl.BlockSpec((bsz, 2), lambda i: (i, 0)),
            fixed(D_UB, 128),
            fixed(D_UB, 128),
            fixed(D_G, 128),
            fixed(128),
            fixed(128, 64),
            fixed(64),
            fixed(64, 1),
            fixed(1),
        ],
        out_specs=pl.BlockSpec((bsz, 1), lambda i: (i, 0)),
        out_shape=jax.ShapeDtypeStruct((BATCH, 1), jnp.float32),
    )(ue, be, ge, mu, mb, qg, W1u, W1b, W1g, b1, W2, b2, W3, b3)
    return out[:, 0]


def kernel(user_idx, book_idx, genre_idx, user_table, book_table, genre_table,
           W1, b1, W2, b2, W3, b3):
    user_idx = user_idx.astype(jnp.int32)
    book_idx = book_idx.astype(jnp.int32)
    genre_idx = genre_idx.astype(jnp.int32)
    up2 = _tc_pack(user_table.T, 4096)
    bp2 = _tc_pack(book_table.T, 4096)
    gp4 = _tc_pack(genre_table.T, 512)
    ue, be, ge = _sc_gather(user_idx, book_idx, genre_idx, up2, bp2, gp4)
    mu = ((user_idx >> 7) & 1).astype(jnp.float32)[:, None]
    mb = ((book_idx >> 7) & 1).astype(jnp.float32)[:, None]
    qq = (genre_idx >> 7) & 3
    qg = jnp.stack([qq & 1, (qq >> 1) & 1], axis=1).astype(jnp.float32)
    return _tc_mlp(ue, be, ge, mu, mb, qg, W1, b1, W2, b2, W3, b3)


# speedup vs baseline: 1.1878x; 1.1878x over previous
import functools

import jax
import jax.numpy as jnp
from jax import lax
from jax.experimental import pallas as pl
from jax.experimental.pallas import tpu as pltpu
from jax.experimental.pallas import tpu_sc as plsc

BATCH = 16384
D_UB = 64
D_G = 32
NC, NS = 2, 16
NW = NC * NS
ROWS_PER_W = BATCH // NW
CHUNK = 128
NCHUNK = ROWS_PER_W // CHUNK


def _pack_body(in_ref, out_ref):
    x = in_ref[...]
    d, bc = x.shape
    k = 128 // d
    xt = x.T
    pieces = []
    for g in range(bc // (128 * k)):
        pieces.append(jnp.concatenate(
            [xt[(g * k + h) * 128:(g * k + h + 1) * 128, :]
             for h in range(k)], axis=1))
    out_ref[...] = jnp.concatenate(pieces, axis=0)


def _tc_pack(tab_t, bc):
    d, n = tab_t.shape
    grid = (n + bc - 1) // bc
    rows = grid * (bc * d // 128)
    return pl.pallas_call(
        _pack_body,
        grid=(grid,),
        in_specs=[pl.BlockSpec((d, bc), lambda i: (0, i))],
        out_specs=pl.BlockSpec((bc * d // 128, 128), lambda i: (i, 0)),
        out_shape=jax.ShapeDtypeStruct((rows, 128), jnp.float32),
    )(tab_t)


def _sc_gather(user_idx, book_idx, genre_idx, up2, bp2, gp4):
    mesh = plsc.VectorSubcoreMesh(
        core_axis_name="c", subcore_axis_name="s", num_cores=NC, num_subcores=NS
    )

    @functools.partial(
        pl.kernel,
        mesh=mesh,
        out_type=[
            jax.ShapeDtypeStruct((BATCH, 128), jnp.float32),
            jax.ShapeDtypeStruct((BATCH, 128), jnp.float32),
            jax.ShapeDtypeStruct((BATCH, 128), jnp.float32),
        ],
        scratch_types=[
            pltpu.VMEM((ROWS_PER_W,), jnp.int32),
            pltpu.VMEM((ROWS_PER_W,), jnp.int32),
            pltpu.VMEM((ROWS_PER_W, 128), jnp.float32),
            pltpu.SemaphoreType.DMA,
        ],
    )
    def gather_kernel(uidx_hbm, bidx_hbm, gidx_hbm, up2_hbm, bp2_hbm, gp4_hbm,
                      uout_hbm, bout_hbm, gout_hbm,
                      idx_v, pidx_v, rows_a, sem):
        wid = lax.axis_index("s") * NC + lax.axis_index("c")
        base = wid * ROWS_PER_W

        def run(idx_hbm, tab_hbm, out_hbm, shift, rows_v):
            pltpu.sync_copy(idx_hbm.at[pl.ds(base, ROWS_PER_W)], idx_v)
            for k in range(ROWS_PER_W // 16):
                s = pl.ds(k * 16, 16)
                i = idx_v[s]
                pidx_v[s] = lax.shift_left(
                    lax.shift_right_logical(i, shift), 7) | (i & 127)
            copies = []
            for j in range(NCHUNK):
                s = pl.ds(j * CHUNK, CHUNK)
                copies.append(pltpu.async_copy(
                    tab_hbm.at[pidx_v.at[s]], rows_v.at[s], sem))
            for c in copies:
                c.wait()
            pltpu.sync_copy(rows_v, out_hbm.at[pl.ds(base, ROWS_PER_W)])

        run(uidx_hbm, up2_hbm, uout_hbm, 8, rows_a)
        run(bidx_hbm, bp2_hbm, bout_hbm, 8, rows_a)
        run(gidx_hbm, gp4_hbm, gout_hbm, 9, rows_a)

    return gather_kernel(user_idx, book_idx, genre_idx, up2, bp2, gp4)


def _mlp_body(ue_ref, be_ref, ge_ref, mu_ref, mb_ref, qg_ref,
              w1u_ref, w1b_ref, w1g_ref, b1_ref, w2_ref, b2_ref, w3_ref,
              b3_ref, out_ref):
    mu = mu_ref[...] > 0.5
    mb = mb_ref[...] > 0.5
    q0 = qg_ref[:, 0:1] > 0.5
    q1 = qg_ref[:, 1:2] > 0.5
    ue = ue_ref[...]
    be = be_ref[...]
    ge = ge_ref[...]
    xu = jnp.where(mu, ue[:, 64:], ue[:, :64])
    xb = jnp.where(mb, be[:, 64:], be[:, :64])
    xg = jnp.where(
        q1,
        jnp.where(q0, ge[:, 96:128], ge[:, 64:96]),
        jnp.where(q0, ge[:, 32:64], ge[:, 0:32]),
    )
    h1 = (xu @ w1u_ref[...] + xb @ w1b_ref[...] + xg @ w1g_ref[...]
          + b1_ref[...])
    h1 = jnp.maximum(h1, 0.0)
    h2 = jnp.maximum(h1 @ w2_ref[...] + b2_ref[...], 0.0)
    out_ref[...] = h2 @ w3_ref[...] + b3_ref[0]


def _tc_mlp(ue, be, ge, mu, mb, qg, W1, b1, W2, b2, W3, b3, bsz=2048):
    W1u, W1b, W1g = W1[:D_UB], W1[D_UB:2 * D_UB], W1[2 * D_UB:]
    grid = BATCH // bsz
    fixed = lambda *shape: pl.BlockSpec(shape, lambda i: (0,) * len(shape))
    out = pl.pallas_call(
        _mlp_body,
        grid=(grid,),
        in_specs=[
            pl.BlockSpec((bsz, 128), lambda i: (i, 0)),
            pl.BlockSpec((bsz, 128), lambda i: (i, 0)),
            pl.BlockSpec((bsz, 128), lambda i: (i, 0)),
            pl.BlockSpec((bsz, 1), lambda i: (i, 0)),
            pl.BlockSpec((bsz, 1), lambda i: (i, 0)),
            pl.BlockSpec((bsz, 2), lambda i: (i, 0)),
            fixed(D_UB, 128),
            fixed(D_UB, 128),
            fixed(D_G, 128),
            fixed(128),
            fixed(128, 64),
            fixed(64),
            fixed(64, 1),
            fixed(1),
        ],
        out_specs=pl.BlockSpec((bsz, 1), lambda i: (i, 0)),
        out_shape=jax.ShapeDtypeStruct((BATCH, 1), jnp.float32),
    )(ue, be, ge, mu, mb, qg, W1u, W1b, W1g, b1, W2, b2, W3, b3)
    return out[:, 0]


def kernel(user_idx, book_idx, genre_idx, user_table, book_table, genre_table,
           W1, b1, W2, b2, W3, b3):
    user_idx = user_idx.astype(jnp.int32)
    book_idx = book_idx.astype(jnp.int32)
    genre_idx = genre_idx.astype(jnp.int32)
    up2 = _tc_pack(user_table.T, 8192)
    bp2 = _tc_pack(book_table.T, 4096)
    gp4 = _tc_pack(genre_table.T, 512)
    ue, be, ge = _sc_gather(user_idx, book_idx, genre_idx, up2, bp2, gp4)
    mu = ((user_idx >> 7) & 1).astype(jnp.float32)[:, None]
    mb = ((book_idx >> 7) & 1).astype(jnp.float32)[:, None]
    qq = (genre_idx >> 7) & 3
    qg = jnp.stack([qq & 1, (qq >> 1) & 1], axis=1).astype(jnp.float32)
    return _tc_mlp(ue, be, ge, mu, mb, qg, W1, b1, W2, b2, W3, b3)


# speedup vs baseline: 1.3163x; 1.1082x over previous
import functools

import jax
import jax.numpy as jnp
from jax import lax
from jax.experimental import pallas as pl
from jax.experimental.pallas import tpu as pltpu
from jax.experimental.pallas import tpu_sc as plsc

BATCH = 16384
D_UB = 64
D_G = 32
NC, NS = 2, 16
NW = NC * NS
ROWS_PER_W = BATCH // NW
CHUNK = 128
NCHUNK = ROWS_PER_W // CHUNK


def _pack_body(in_ref, out_ref):
    x = in_ref[...]
    d, bc = x.shape
    k = 128 // d
    xt = x.T
    pieces = []
    for g in range(bc // (128 * k)):
        pieces.append(jnp.concatenate(
            [xt[(g * k + h) * 128:(g * k + h + 1) * 128, :]
             for h in range(k)], axis=1))
    out_ref[...] = jnp.concatenate(pieces, axis=0)


def _tc_pack(tab_t, bc):
    d, n = tab_t.shape
    grid = (n + bc - 1) // bc
    rows = grid * (bc * d // 128)
    return pl.pallas_call(
        _pack_body,
        grid=(grid,),
        in_specs=[pl.BlockSpec((d, bc), lambda i: (0, i))],
        out_specs=pl.BlockSpec((bc * d // 128, 128), lambda i: (i, 0)),
        out_shape=jax.ShapeDtypeStruct((rows, 128), jnp.float32),
    )(tab_t)


def _sc_gather(user_idx, book_idx, genre_idx, up2, bp2, gp4):
    mesh = plsc.VectorSubcoreMesh(
        core_axis_name="c", subcore_axis_name="s", num_cores=NC, num_subcores=NS
    )

    @functools.partial(
        pl.kernel,
        mesh=mesh,
        out_type=[
            jax.ShapeDtypeStruct((BATCH, 128), jnp.float32),
            jax.ShapeDtypeStruct((BATCH, 128), jnp.float32),
            jax.ShapeDtypeStruct((BATCH, 128), jnp.float32),
        ],
        scratch_types=[
            pltpu.VMEM((ROWS_PER_W,), jnp.int32),
            pltpu.VMEM((ROWS_PER_W,), jnp.int32),
            pltpu.VMEM((ROWS_PER_W, 128), jnp.float32),
            pltpu.SemaphoreType.DMA,
        ],
    )
    def gather_kernel(uidx_hbm, bidx_hbm, gidx_hbm, up2_hbm, bp2_hbm, gp4_hbm,
                      uout_hbm, bout_hbm, gout_hbm,
                      idx_v, pidx_v, rows_a, sem):
        wid = lax.axis_index("s") * NC + lax.axis_index("c")
        base = wid * ROWS_PER_W

        def run(idx_hbm, tab_hbm, out_hbm, shift, rows_v):
            pltpu.sync_copy(idx_hbm.at[pl.ds(base, ROWS_PER_W)], idx_v)
            for k in range(ROWS_PER_W // 16):
                s = pl.ds(k * 16, 16)
                i = idx_v[s]
                pidx_v[s] = lax.shift_left(
                    lax.shift_right_logical(i, shift), 7) | (i & 127)
            copies = []
            for j in range(NCHUNK):
                s = pl.ds(j * CHUNK, CHUNK)
                copies.append(pltpu.async_copy(
                    tab_hbm.at[pidx_v.at[s]], rows_v.at[s], sem))
            for c in copies:
                c.wait()
            pltpu.sync_copy(rows_v, out_hbm.at[pl.ds(base, ROWS_PER_W)])

        run(uidx_hbm, up2_hbm, uout_hbm, 8, rows_a)
        run(bidx_hbm, bp2_hbm, bout_hbm, 8, rows_a)
        run(gidx_hbm, gp4_hbm, gout_hbm, 9, rows_a)

    return gather_kernel(user_idx, book_idx, genre_idx, up2, bp2, gp4)


def _mlp_body(ue_ref, be_ref, ge_ref, mu_ref, mb_ref, qg_ref,
              w1u_ref, w1b_ref, w1g_ref, b1_ref, w2_ref, b2_ref, w3_ref,
              b3_ref, out_ref):
    mu = mu_ref[...] > 0.5
    mb = mb_ref[...] > 0.5
    q0 = qg_ref[:, 0:1] > 0.5
    q1 = qg_ref[:, 1:2] > 0.5
    ue = ue_ref[...]
    be = be_ref[...]
    ge = ge_ref[...]
    xu = jnp.where(mu, ue[:, 64:], ue[:, :64])
    xb = jnp.where(mb, be[:, 64:], be[:, :64])
    xg = jnp.where(
        q1,
        jnp.where(q0, ge[:, 96:128], ge[:, 64:96]),
        jnp.where(q0, ge[:, 32:64], ge[:, 0:32]),
    )
    h1 = (xu @ w1u_ref[...] + xb @ w1b_ref[...] + xg @ w1g_ref[...]
          + b1_ref[...])
    h1 = jnp.maximum(h1, 0.0)
    h2 = jnp.maximum(h1 @ w2_ref[...] + b2_ref[...], 0.0)
    out_ref[...] = h2 @ w3_ref[...] + b3_ref[0]


def _tc_mlp(ue, be, ge, mu, mb, qg, W1, b1, W2, b2, W3, b3, bsz=2048):
    W1u, W1b, W1g = W1[:D_UB], W1[D_UB:2 * D_UB], W1[2 * D_UB:]
    grid = BATCH // bsz
    fixed = lambda *shape: pl.BlockSpec(shape, lambda i: (0,) * len(shape))
    out = pl.pallas_call(
        _mlp_body,
        grid=(grid,),
        in_specs=[
            pl.BlockSpec((bsz, 128), lambda i: (i, 0)),
            pl.BlockSpec((bsz, 128), lambda i: (i, 0)),
            pl.BlockSpec((bsz, 128), lambda i: (i, 0)),
            pl.BlockSpec((bsz, 1), lambda i: (i, 0)),
            pl.BlockSpec((bsz, 1), lambda i: (i, 0)),
            pl.BlockSpec((bsz, 2), lambda i: (i, 0)),
            fixed(D_UB, 128),
            fixed(D_UB, 128),
            fixed(D_G, 128),
            fixed(128),
            fixed(128, 64),
            fixed(64),
            fixed(64, 1),
            fixed(1),
        ],
        out_specs=pl.BlockSpec((bsz, 1), lambda i: (i, 0)),
        out_shape=jax.ShapeDtypeStruct((BATCH, 1), jnp.float32),
    )(ue, be, ge, mu, mb, qg, W1u, W1b, W1g, b1, W2, b2, W3, b3)
    return out[:, 0]


def kernel(user_idx, book_idx, genre_idx, user_table, book_table, genre_table,
           W1, b1, W2, b2, W3, b3):
    user_idx = user_idx.astype(jnp.int32)
    book_idx = book_idx.astype(jnp.int32)
    genre_idx = genre_idx.astype(jnp.int32)
    up2 = _tc_pack(user_table.T, 16384)
    bp2 = _tc_pack(book_table.T, 8192)
    gp4 = _tc_pack(genre_table.T, 512)
    ue, be, ge = _sc_gather(user_idx, book_idx, genre_idx, up2, bp2, gp4)
    mu = ((user_idx >> 7) & 1).astype(jnp.float32)[:, None]
    mb = ((book_idx >> 7) & 1).astype(jnp.float32)[:, None]
    qq = (genre_idx >> 7) & 3
    qg = jnp.stack([qq & 1, (qq >> 1) & 1], axis=1).astype(jnp.float32)
    return _tc_mlp(ue, be, ge, mu, mb, qg, W1, b1, W2, b2, W3, b3)


# speedup vs baseline: 1.3671x; 1.0386x over previous
import functools

import jax
import jax.numpy as jnp
from jax import lax
from jax.experimental import pallas as pl
from jax.experimental.pallas import tpu as pltpu
from jax.experimental.pallas import tpu_sc as plsc

BATCH = 16384
D_UB = 64
D_G = 32
NC, NS = 2, 16
NW = NC * NS
ROWS_PER_W = BATCH // NW
CHUNK = 128
NCHUNK = ROWS_PER_W // CHUNK


def _pack_body(in_ref, out_ref):
    x = in_ref[...]
    d, bc = x.shape
    k = 128 // d
    xt = x.T
    pieces = []
    for g in range(bc // (128 * k)):
        pieces.append(jnp.concatenate(
            [xt[(g * k + h) * 128:(g * k + h + 1) * 128, :]
             for h in range(k)], axis=1))
    out_ref[...] = jnp.concatenate(pieces, axis=0)


def _tc_pack(tab_t, bc):
    d, n = tab_t.shape
    grid = (n + bc - 1) // bc
    rows = grid * (bc * d // 128)
    return pl.pallas_call(
        _pack_body,
        grid=(grid,),
        in_specs=[pl.BlockSpec((d, bc), lambda i: (0, i))],
        out_specs=pl.BlockSpec((bc * d // 128, 128), lambda i: (i, 0)),
        out_shape=jax.ShapeDtypeStruct((rows, 128), jnp.float32),
    )(tab_t)


def _sc_gather(user_idx, book_idx, genre_idx, up2, bp2, gp4):
    mesh = plsc.VectorSubcoreMesh(
        core_axis_name="c", subcore_axis_name="s", num_cores=NC, num_subcores=NS
    )

    @functools.partial(
        pl.kernel,
        mesh=mesh,
        out_type=[
            jax.ShapeDtypeStruct((BATCH, 128), jnp.float32),
            jax.ShapeDtypeStruct((BATCH, 128), jnp.float32),
            jax.ShapeDtypeStruct((BATCH, 128), jnp.float32),
        ],
        scratch_types=[
            pltpu.VMEM((ROWS_PER_W,), jnp.int32),
            pltpu.VMEM((ROWS_PER_W,), jnp.int32),
            pltpu.VMEM((ROWS_PER_W, 128), jnp.float32),
            pltpu.SemaphoreType.DMA,
        ],
    )
    def gather_kernel(uidx_hbm, bidx_hbm, gidx_hbm, up2_hbm, bp2_hbm, gp4_hbm,
                      uout_hbm, bout_hbm, gout_hbm,
                      idx_v, pidx_v, rows_a, sem):
        wid = lax.axis_index("s") * NC + lax.axis_index("c")
        base = wid * ROWS_PER_W

        def run(idx_hbm, tab_hbm, out_hbm, shift, rows_v):
            pltpu.sync_copy(idx_hbm.at[pl.ds(base, ROWS_PER_W)], idx_v)
            for k in range(ROWS_PER_W // 16):
                s = pl.ds(k * 16, 16)
                i = idx_v[s]
                pidx_v[s] = lax.shift_left(
                    lax.shift_right_logical(i, shift), 7) | (i & 127)
            copies = []
            for j in range(NCHUNK):
                s = pl.ds(j * CHUNK, CHUNK)
                copies.append(pltpu.async_copy(
                    tab_hbm.at[pidx_v.at[s]], rows_v.at[s], sem))
            for c in copies:
                c.wait()
            pltpu.sync_copy(rows_v, out_hbm.at[pl.ds(base, ROWS_PER_W)])

        run(uidx_hbm, up2_hbm, uout_hbm, 8, rows_a)
        run(bidx_hbm, bp2_hbm, bout_hbm, 8, rows_a)
        run(gidx_hbm, gp4_hbm, gout_hbm, 9, rows_a)

    return gather_kernel(user_idx, book_idx, genre_idx, up2, bp2, gp4)


def _mlp_body(ue_ref, be_ref, ge_ref, mu_ref, mb_ref, qg_ref,
              w1u_ref, w1b_ref, w1g_ref, b1_ref, w2_ref, b2_ref, w3_ref,
              b3_ref, out_ref):
    mu = mu_ref[...] > 0.5
    mb = mb_ref[...] > 0.5
    q0 = qg_ref[:, 0:1] > 0.5
    q1 = qg_ref[:, 1:2] > 0.5
    ue = ue_ref[...]
    be = be_ref[...]
    ge = ge_ref[...]
    xu = jnp.where(mu, ue[:, 64:], ue[:, :64])
    xb = jnp.where(mb, be[:, 64:], be[:, :64])
    xg = jnp.where(
        q1,
        jnp.where(q0, ge[:, 96:128], ge[:, 64:96]),
        jnp.where(q0, ge[:, 32:64], ge[:, 0:32]),
    )
    h1 = (xu @ w1u_ref[...] + xb @ w1b_ref[...] + xg @ w1g_ref[...]
          + b1_ref[...])
    h1 = jnp.maximum(h1, 0.0)
    h2 = jnp.maximum(h1 @ w2_ref[...] + b2_ref[...], 0.0)
    out_ref[...] = h2 @ w3_ref[...] + b3_ref[0]


def _tc_mlp(ue, be, ge, mu, mb, qg, W1, b1, W2, b2, W3, b3, bsz=2048):
    W1u, W1b, W1g = W1[:D_UB], W1[D_UB:2 * D_UB], W1[2 * D_UB:]
    grid = BATCH // bsz
    fixed = lambda *shape: pl.BlockSpec(shape, lambda i: (0,) * len(shape))
    out = pl.pallas_call(
        _mlp_body,
        grid=(grid,),
        in_specs=[
            pl.BlockSpec((bsz, 128), lambda i: (i, 0)),
            pl.BlockSpec((bsz, 128), lambda i: (i, 0)),
            pl.BlockSpec((bsz, 128), lambda i: (i, 0)),
            pl.BlockSpec((bsz, 1), lambda i: (i, 0)),
            pl.BlockSpec((bsz, 1), lambda i: (i, 0)),
            pl.BlockSpec((bsz, 2), lambda i: (i, 0)),
            fixed(D_UB, 128),
            fixed(D_UB, 128),
            fixed(D_G, 128),
            fixed(128),
            fixed(128, 64),
            fixed(64),
            fixed(64, 1),
            fixed(1),
        ],
        out_specs=pl.BlockSpec((bsz, 1), lambda i: (i, 0)),
        out_shape=jax.ShapeDtypeStruct((BATCH, 1), jnp.float32),
    )(ue, be, ge, mu, mb, qg, W1u, W1b, W1g, b1, W2, b2, W3, b3)
    return out[:, 0]


def kernel(user_idx, book_idx, genre_idx, user_table, book_table, genre_table,
           W1, b1, W2, b2, W3, b3):
    user_idx = user_idx.astype(jnp.int32)
    book_idx = book_idx.astype(jnp.int32)
    genre_idx = genre_idx.astype(jnp.int32)
    up2 = _tc_pack(user_table.T, 32768)
    bp2 = _tc_pack(book_table.T, 8192)
    gp4 = _tc_pack(genre_table.T, 512)
    ue, be, ge = _sc_gather(user_idx, book_idx, genre_idx, up2, bp2, gp4)
    mu = ((user_idx >> 7) & 1).astype(jnp.float32)[:, None]
    mb = ((book_idx >> 7) & 1).astype(jnp.float32)[:, None]
    qq = (genre_idx >> 7) & 3
    qg = jnp.stack([qq & 1, (qq >> 1) & 1], axis=1).astype(jnp.float32)
    return _tc_mlp(ue, be, ge, mu, mb, qg, W1, b1, W2, b2, W3, b3)
